# baseline (speedup 1.0000x reference)
import functools

import jax
import jax.numpy as jnp
from jax import lax
from jax.experimental import pallas as pl
from jax.experimental.pallas import tpu as pltpu
from jax.experimental.pallas import tpu_sc as plsc

NUM_NODES = 100000
FEAT = 128
NUM_GRAPHS = 64

NUM_CORES = 2
NUM_SUBCORES = 16
NUM_WORKERS = NUM_CORES * NUM_SUBCORES
CHUNK = 4096
ROWS_PER_CHUNK = CHUNK // 128
PAD_N = NUM_WORKERS * CHUNK
PAD_VALUE = NUM_GRAPHS
HIST = 128

BLOCK_ROWS = 2000
GRID = NUM_NODES // BLOCK_ROWS


def _sc_hist_body(batch_ref, out_ref, idx_v, ones_v, zeros_v, shared):
    cid = lax.axis_index("c")
    sid = lax.axis_index("s")
    wid = sid * NUM_CORES + cid

    for j in range(HIST // 16):
        zeros_v[pl.ds(j * 16, 16)] = jnp.zeros((16,), jnp.float32)
    for j in range(128 // 16):
        ones_v[pl.ds(j * 16, 16)] = jnp.ones((16,), jnp.float32)

    @pl.when(sid == 0)
    def _():
        pltpu.sync_copy(zeros_v, shared)

    plsc.subcore_barrier()

    pltpu.sync_copy(batch_ref.at[pl.ds(wid * ROWS_PER_CHUNK, ROWS_PER_CHUNK)], idx_v)

    def _scatter(j, carry):
        pltpu.sync_copy(ones_v, shared.at[idx_v.at[j]], add=True)
        return carry

    lax.fori_loop(0, ROWS_PER_CHUNK, _scatter, 0)

    plsc.subcore_barrier()

    @pl.when(sid == 0)
    def _():
        pltpu.sync_copy(shared, out_ref.at[cid])


@functools.cache
def _sc_hist():
    return pl.kernel(
        _sc_hist_body,
        out_type=jax.ShapeDtypeStruct((NUM_CORES, HIST), jnp.float32),
        mesh=plsc.VectorSubcoreMesh(core_axis_name="c", subcore_axis_name="s"),
        scratch_types=[
            pltpu.VMEM((ROWS_PER_CHUNK, 128), jnp.int32),
            pltpu.VMEM((128,), jnp.float32),
            pltpu.VMEM((HIST,), jnp.float32),
            pltpu.VMEM_SHARED((HIST,), jnp.float32),
        ],
    )


def _tc_scale_body(deg_ref, batch_ref, x_ref, o_ref):
    deg = deg_ref[0:1, :] + deg_ref[1:2, :]
    inv = jnp.where(deg > 0.0, lax.rsqrt(deg), 0.0)
    ids = batch_ref[0]
    gid = lax.broadcasted_iota(jnp.int32, (BLOCK_ROWS, HIST), 1)
    onehot = (ids == gid).astype(jnp.float32)
    scale = jnp.dot(
        onehot, inv.reshape(HIST, 1), preferred_element_type=jnp.float32
    )
    o_ref[...] = x_ref[...] * scale


def kernel(x, batch):
    batch = batch.astype(jnp.int32)
    pad = jnp.full((PAD_N - NUM_NODES,), PAD_VALUE, jnp.int32)
    batch2d = jnp.concatenate([batch, pad]).reshape(PAD_N // 128, 128)
    deg2 = _sc_hist()(batch2d)

    batch3 = batch.reshape(GRID, BLOCK_ROWS, 1)
    return pl.pallas_call(
        _tc_scale_body,
        grid=(GRID,),
        in_specs=[
            pl.BlockSpec((NUM_CORES, HIST), lambda i: (0, 0)),
            pl.BlockSpec((1, BLOCK_ROWS, 1), lambda i: (i, 0, 0)),
            pl.BlockSpec((BLOCK_ROWS, FEAT), lambda i: (i, 0)),
        ],
        out_specs=pl.BlockSpec((BLOCK_ROWS, FEAT), lambda i: (i, 0)),
        out_shape=jax.ShapeDtypeStruct((NUM_NODES, FEAT), jnp.float32),
        compiler_params=pltpu.CompilerParams(
            dimension_semantics=("arbitrary",),
        ),
    )(deg2, batch3, x)


# speedup vs baseline: 1.1271x; 1.1271x over previous
import functools

import jax
import jax.numpy as jnp
from jax import lax
from jax.experimental import pallas as pl
from jax.experimental.pallas import tpu as pltpu
from jax.experimental.pallas import tpu_sc as plsc

NUM_NODES = 100000
FEAT = 128
NUM_GRAPHS = 64

NUM_CORES = 2
NUM_SUBCORES = 16
NUM_WORKERS = NUM_CORES * NUM_SUBCORES
CHUNK = 4096
ROWS_PER_CHUNK = CHUNK // 128
PAD_N = NUM_WORKERS * CHUNK
PAD_VALUE = NUM_GRAPHS
HIST = 128

BLOCK_ROWS = 4000
GRID = NUM_NODES // BLOCK_ROWS


def _sc_hist_body(batch_ref, ones_ref, out_ref, idx_v, ones_v, zeros_v, shared, sem):
    cid = lax.axis_index("c")
    sid = lax.axis_index("s")
    wid = sid * NUM_CORES + cid

    load_idx = pltpu.async_copy(batch_ref.at[pl.ds(wid * CHUNK, CHUNK)], idx_v, sem)
    load_ones = pltpu.async_copy(ones_ref, ones_v, sem)

    for j in range(HIST // 16):
        zeros_v[pl.ds(j * 16, 16)] = jnp.zeros((16,), jnp.float32)

    @pl.when(sid == 0)
    def _():
        pltpu.sync_copy(zeros_v, shared)

    load_idx.wait()
    load_ones.wait()
    plsc.subcore_barrier()

    pltpu.sync_copy(ones_v, shared.at[idx_v], add=True)

    plsc.subcore_barrier()

    @pl.when(sid == 0)
    def _():
        pltpu.sync_copy(shared, out_ref.at[cid])


@functools.cache
def _sc_hist():
    return pl.kernel(
        _sc_hist_body,
        out_type=jax.ShapeDtypeStruct((NUM_CORES, HIST), jnp.float32),
        mesh=plsc.VectorSubcoreMesh(core_axis_name="c", subcore_axis_name="s"),
        scratch_types=[
            pltpu.VMEM((CHUNK,), jnp.int32),
            pltpu.VMEM((CHUNK,), jnp.float32),
            pltpu.VMEM((HIST,), jnp.float32),
            pltpu.VMEM_SHARED((HIST,), jnp.float32),
            pltpu.SemaphoreType.DMA,
        ],
    )


def _tc_scale_body(deg_ref, batch_ref, x_ref, o_ref):
    deg = deg_ref[0:1, :] + deg_ref[1:2, :]
    inv = jnp.where(deg > 0.0, lax.rsqrt(deg), 0.0)
    ids = batch_ref[0]
    gid = lax.broadcasted_iota(jnp.int32, (BLOCK_ROWS, HIST), 1)
    onehot = (ids == gid).astype(jnp.float32)
    scale = jnp.dot(
        onehot, inv.reshape(HIST, 1), preferred_element_type=jnp.float32
    )
    o_ref[...] = x_ref[...] * scale


def kernel(x, batch):
    batch = batch.astype(jnp.int32)
    pad = jnp.full((PAD_N - NUM_NODES,), PAD_VALUE, jnp.int32)
    batch1d = jnp.concatenate([batch, pad])
    ones1d = jnp.ones((CHUNK,), jnp.float32)
    deg2 = _sc_hist()(batch1d, ones1d)

    batch3 = batch.reshape(GRID, BLOCK_ROWS, 1)
    return pl.pallas_call(
        _tc_scale_body,
        grid=(GRID,),
        in_specs=[
            pl.BlockSpec((NUM_CORES, HIST), lambda i: (0, 0)),
            pl.BlockSpec((1, BLOCK_ROWS, 1), lambda i: (i, 0, 0)),
            pl.BlockSpec((BLOCK_ROWS, FEAT), lambda i: (i, 0)),
        ],
        out_specs=pl.BlockSpec((BLOCK_ROWS, FEAT), lambda i: (i, 0)),
        out_shape=jax.ShapeDtypeStruct((NUM_NODES, FEAT), jnp.float32),
        compiler_params=pltpu.CompilerParams(
            dimension_semantics=("arbitrary",),
        ),
    )(deg2, batch3, x)


# speedup vs baseline: 1.1618x; 1.0308x over previous
import functools

import jax
import jax.numpy as jnp
from jax import lax
from jax.experimental import pallas as pl
from jax.experimental.pallas import tpu as pltpu
from jax.experimental.pallas import tpu_sc as plsc

NUM_NODES = 100000
FEAT = 128
NUM_GRAPHS = 64

NUM_CORES = 2
NUM_SUBCORES = 16
NUM_WORKERS = NUM_CORES * NUM_SUBCORES
CHUNK = 3200
PAD_N = NUM_WORKERS * CHUNK
PAD_VALUE = NUM_GRAPHS
HIST = 128

BLOCK_ROWS = 10000
GRID = NUM_NODES // BLOCK_ROWS


def _sc_hist_body(batch_ref, out_ref, idx_v, ones_v, zeros_v, shared, sem):
    cid = lax.axis_index("c")
    sid = lax.axis_index("s")
    wid = sid * NUM_CORES + cid

    load_idx = pltpu.async_copy(batch_ref.at[pl.ds(wid * CHUNK, CHUNK)], idx_v, sem)

    ones16 = jnp.ones((16,), jnp.float32)

    def _fill(j, carry):
        ones_v[pl.ds(j * 16, 16)] = ones16
        return carry

    lax.fori_loop(0, CHUNK // 16, _fill, 0)
    for j in range(HIST // 16):
        zeros_v[pl.ds(j * 16, 16)] = jnp.zeros((16,), jnp.float32)

    @pl.when(sid == 0)
    def _():
        pltpu.sync_copy(zeros_v, shared)

    load_idx.wait()
    plsc.subcore_barrier()

    pltpu.sync_copy(ones_v, shared.at[idx_v], add=True)

    plsc.subcore_barrier()

    @pl.when(sid == 0)
    def _():
        pltpu.sync_copy(shared, out_ref.at[cid])


@functools.cache
def _sc_hist():
    return pl.kernel(
        _sc_hist_body,
        out_type=jax.ShapeDtypeStruct((NUM_CORES, HIST), jnp.float32),
        mesh=plsc.VectorSubcoreMesh(core_axis_name="c", subcore_axis_name="s"),
        scratch_types=[
            pltpu.VMEM((CHUNK,), jnp.int32),
            pltpu.VMEM((CHUNK,), jnp.float32),
            pltpu.VMEM((HIST,), jnp.float32),
            pltpu.VMEM_SHARED((HIST,), jnp.float32),
            pltpu.SemaphoreType.DMA,
        ],
    )


def _tc_scale_body(deg_ref, batch_ref, x_ref, o_ref):
    deg = deg_ref[0:1, :] + deg_ref[1:2, :]
    inv = jnp.where(deg > 0.0, lax.rsqrt(deg), 0.0)
    ids = batch_ref[0]
    gid = lax.broadcasted_iota(jnp.int32, (BLOCK_ROWS, HIST), 1)
    onehot = (ids == gid).astype(jnp.float32)
    scale = jnp.dot(
        onehot, inv.reshape(HIST, 1), preferred_element_type=jnp.float32
    )
    o_ref[...] = x_ref[...] * scale


def kernel(x, batch):
    batch = batch.astype(jnp.int32)
    pad = jnp.full((PAD_N - NUM_NODES,), PAD_VALUE, jnp.int32)
    batch1d = jnp.concatenate([batch, pad])
    deg2 = _sc_hist()(batch1d)

    batch3 = batch.reshape(GRID, BLOCK_ROWS, 1)
    return pl.pallas_call(
        _tc_scale_body,
        grid=(GRID,),
        in_specs=[
            pl.BlockSpec((NUM_CORES, HIST), lambda i: (0, 0)),
            pl.BlockSpec((1, BLOCK_ROWS, 1), lambda i: (i, 0, 0)),
            pl.BlockSpec((BLOCK_ROWS, FEAT), lambda i: (i, 0)),
        ],
        out_specs=pl.BlockSpec((BLOCK_ROWS, FEAT), lambda i: (i, 0)),
        out_shape=jax.ShapeDtypeStruct((NUM_NODES, FEAT), jnp.float32),
        compiler_params=pltpu.CompilerParams(
            dimension_semantics=("arbitrary",),
        ),
    )(deg2, batch3, x)
